# baseline (speedup 1.0000x reference)
import jax
import jax.numpy as jnp
from jax import lax
from jax.experimental import pallas as pl
from jax.experimental.pallas import tpu as pltpu
from jax.experimental.pallas import tpu_sc as plsc

N_NODES = 10000
N_EDGES = 320000
D_FEAT = 128
HALF = 144

NS = 16

N_PAD = 10240
E_PAD = 327680

CH = 128
N_CHUNKS = E_PAD // CH
TC_CHUNKS = N_CHUNKS // NS
N_ACC = 10112
STRIPE = N_ACC // NS

MB = 1024
EB = 4000

_SC_PARAMS = pltpu.CompilerParams(use_tc_tiling_on_sc=False)


def _dense_pre_body(x_ref, w1_ref, b1_ref, w2_ref, b2_ref,
                    wcat_a_ref, wcat_b_ref, wx_a_ref, wx_b_ref, bias_b_ref,
                    h2a_ref, h2b_ref, xwa_ref, xwb_ref):
    x = x_ref[...]
    h1 = jnp.maximum(x @ w1_ref[...] + b1_ref[...], 0.0)
    h = h1 @ w2_ref[...] + b2_ref[...]
    h2a_ref[...] = h @ wcat_a_ref[...]
    h2b_ref[...] = h @ wcat_b_ref[...]
    xwa_ref[...] = x @ wx_a_ref[...]
    xwb_ref[...] = x @ wx_b_ref[...] + bias_b_ref[...]


def _dense_pre(x, w1, b1, w2, b2, wcat_a, wcat_b, wx_a, wx_b, bias_b):
    grid = (N_PAD // MB,)
    full = lambda shape: pl.BlockSpec(shape, lambda i: (0, 0))
    return pl.pallas_call(
        _dense_pre_body,
        grid=grid,
        in_specs=[
            pl.BlockSpec((MB, D_FEAT), lambda i: (i, 0)),
            full((D_FEAT, 128)), full((1, 128)),
            full((128, 512)), full((1, 512)),
            full((512, HALF)), full((512, HALF)),
            full((D_FEAT, HALF)), full((D_FEAT, HALF)), full((1, HALF)),
        ],
        out_specs=[pl.BlockSpec((MB, HALF), lambda i: (i, 0))] * 4,
        out_shape=[jax.ShapeDtypeStruct((N_PAD, HALF), jnp.float32)] * 4,
    )(x, w1, b1, w2, b2, wcat_a, wcat_b, wx_a, wx_b, bias_b)


IB = 8


def _sc_segsum_body(h2_hbm, xw_hbm, src_hbm, dst_hbm, p_hbm,
                    idx_s, idx_d, rows0, rows1, sem0, sem1, acc):
    s = lax.axis_index("s")

    pltpu.sync_copy(xw_hbm.at[pl.ds(s * STRIPE, STRIPE)],
                    acc.at[pl.ds(s * STRIPE, STRIPE)])
    plsc.subcore_barrier()

    def gather(j, buf, sem):
        return pltpu.async_copy(h2_hbm.at[idx_s.at[j]], buf, sem)

    def block(g, _):
        row0 = s * TC_CHUNKS + g * IB
        pltpu.sync_copy(src_hbm.at[pl.ds(row0, IB)], idx_s)
        pltpu.sync_copy(dst_hbm.at[pl.ds(row0, IB)], idx_d)
        gather(0, rows0, sem0)

        def pair(k, _):
            j0 = 2 * k
            gather(j0 + 1, rows1, sem1)
            pltpu.make_async_copy(h2_hbm.at[idx_s.at[j0]], rows0, sem0).wait()
            pltpu.sync_copy(rows0, acc.at[idx_d.at[j0]], add=True)

            @pl.when(j0 + 2 < IB)
            def _():
                gather(j0 + 2, rows0, sem0)

            pltpu.make_async_copy(h2_hbm.at[idx_s.at[j0]], rows1, sem1).wait()
            pltpu.sync_copy(rows1, acc.at[idx_d.at[j0 + 1]], add=True)
            return 0

        lax.fori_loop(0, IB // 2, pair, 0)
        return 0

    lax.fori_loop(0, TC_CHUNKS // IB, block, 0)
    plsc.subcore_barrier()

    pltpu.sync_copy(acc.at[pl.ds(s * STRIPE, STRIPE)],
                    p_hbm.at[pl.ds(s * STRIPE, STRIPE)])


def _sc_segsum(h2, xw, src2d, dst2d):
    mesh = plsc.VectorSubcoreMesh(core_axis_name="c", subcore_axis_name="s",
                                  num_cores=1, num_subcores=NS)
    return pl.kernel(
        _sc_segsum_body,
        compiler_params=_SC_PARAMS,
        out_type=jax.ShapeDtypeStruct((N_ACC, HALF), jnp.float32),
        mesh=mesh,
        scratch_types=[
            pltpu.VMEM((IB, CH), jnp.int32),
            pltpu.VMEM((IB, CH), jnp.int32),
            pltpu.VMEM((CH, HALF), jnp.float32),
            pltpu.VMEM((CH, HALF), jnp.float32),
            pltpu.SemaphoreType.DMA,
            pltpu.SemaphoreType.DMA,
            pltpu.VMEM_SHARED((N_ACC, HALF), jnp.float32),
        ],
    )(h2, xw, src2d, dst2d)


CHE = 128
IBE = 16
EH = E_PAD // 2
EH_CHUNKS = EH // CHE
TE_CHUNKS = EH_CHUNKS // NS


def _sc_edge_body(base_chunk,
                  pa_hbm, pb_hbm, src_hbm, dst_hbm,
                  th0_hbm, th1_hbm, u_hbm,
                  idx_s, idx_d, ra0, ra1, rb0, rb1, uv,
                  sa0, sa1, sb0, sb1):
    s = lax.axis_index("s")

    def block(g, _):
        row0 = s * TE_CHUNKS + g * IBE
        pltpu.sync_copy(src_hbm.at[pl.ds(base_chunk + row0, IBE)], idx_s)
        pltpu.sync_copy(dst_hbm.at[pl.ds(base_chunk + row0, IBE)], idx_d)

        def issue(j, ra, sa, rb, sb):
            pltpu.async_copy(pa_hbm.at[idx_s.at[j]], ra, sa)
            pltpu.async_copy(pb_hbm.at[idx_d.at[j]], rb, sb)

        def consume(j, ra, sa, rb, sb):
            base = (row0 + j) * CHE
            pltpu.make_async_copy(pa_hbm.at[idx_s.at[j]], ra, sa).wait()
            pltpu.make_async_copy(pb_hbm.at[idx_d.at[j]], rb, sb).wait()
            pltpu.sync_copy(ra.at[:, pl.ds(0, 128)], th0_hbm.at[pl.ds(base, CHE)])
            pltpu.sync_copy(rb.at[:, pl.ds(0, 128)], th1_hbm.at[pl.ds(base, CHE)])

            def uadd(e, _):
                uv[e, :] = ra[e, pl.ds(128, 16)] + rb[e, pl.ds(128, 16)]
                return 0

            lax.fori_loop(0, CHE, uadd, 0)
            pltpu.sync_copy(uv, u_hbm.at[pl.ds(base, CHE)])

        issue(0, ra0, sa0, rb0, sb0)

        def pair(k, _):
            j0 = 2 * k
            issue(j0 + 1, ra1, sa1, rb1, sb1)
            consume(j0, ra0, sa0, rb0, sb0)

            @pl.when(j0 + 2 < IBE)
            def _():
                issue(j0 + 2, ra0, sa0, rb0, sb0)

            consume(j0 + 1, ra1, sa1, rb1, sb1)
            return 0

        lax.fori_loop(0, IBE // 2, pair, 0)
        return 0

    lax.fori_loop(0, TE_CHUNKS // IBE, block, 0)


def _sc_edge(pa, pb, src2d, dst2d, base_chunk):
    import functools
    mesh = plsc.VectorSubcoreMesh(core_axis_name="c", subcore_axis_name="s",
                                  num_cores=1, num_subcores=NS)
    return pl.kernel(
        functools.partial(_sc_edge_body, base_chunk),
        compiler_params=_SC_PARAMS,
        out_type=[jax.ShapeDtypeStruct((EH, 128), jnp.float32),
                  jax.ShapeDtypeStruct((EH, 128), jnp.float32),
                  jax.ShapeDtypeStruct((EH, 16), jnp.float32)],
        mesh=mesh,
        scratch_types=[
            pltpu.VMEM((IBE, CHE), jnp.int32),
            pltpu.VMEM((IBE, CHE), jnp.int32),
            pltpu.VMEM((CHE, HALF), jnp.float32),
            pltpu.VMEM((CHE, HALF), jnp.float32),
            pltpu.VMEM((CHE, HALF), jnp.float32),
            pltpu.VMEM((CHE, HALF), jnp.float32),
            pltpu.VMEM((CHE, 16), jnp.float32),
            pltpu.SemaphoreType.DMA,
            pltpu.SemaphoreType.DMA,
            pltpu.SemaphoreType.DMA,
            pltpu.SemaphoreType.DMA,
        ],
    )(pa, pb, src2d, dst2d)


FB = 2560


def _final_body(t0_ref, t1_ref, u_ref, vh_ref, o_ref):
    hid = jnp.maximum(t0_ref[...] + t1_ref[...], 0.0)
    o_ref[...] = hid @ vh_ref[...] + u_ref[...]


def _final_half1(t0, t1, u, vh):
    return pl.pallas_call(
        _final_body,
        grid=(EH // FB,),
        in_specs=[
            pl.BlockSpec((FB, 128), lambda i: (i, 0)),
            pl.BlockSpec((FB, 128), lambda i: (i, 0)),
            pl.BlockSpec((FB, 16), lambda i: (i, 0)),
            pl.BlockSpec((128, 16), lambda i: (0, 0)),
        ],
        out_specs=pl.BlockSpec((FB, 16), lambda i: (i, 0)),
        out_shape=jax.ShapeDtypeStruct((N_EDGES, 16), jnp.float32),
    )(t0, t1, u, vh)


def _final_half2_body(t0_ref, t1_ref, u_ref, vh_ref, prev_ref, o_ref):
    del prev_ref
    hid = jnp.maximum(t0_ref[...] + t1_ref[...], 0.0)
    o_ref[...] = hid @ vh_ref[...] + u_ref[...]


def _final_half2(t0, t1, u, vh, prev):
    n_blocks = (N_EDGES - EH) // FB
    base = EH // FB
    return pl.pallas_call(
        _final_half2_body,
        grid=(n_blocks,),
        in_specs=[
            pl.BlockSpec((FB, 128), lambda i: (i, 0)),
            pl.BlockSpec((FB, 128), lambda i: (i, 0)),
            pl.BlockSpec((FB, 16), lambda i: (i, 0)),
            pl.BlockSpec((128, 16), lambda i: (0, 0)),
            pl.BlockSpec(memory_space=pl.ANY),
        ],
        out_specs=pl.BlockSpec((FB, 16), lambda i: (base + i, 0)),
        out_shape=jax.ShapeDtypeStruct((N_EDGES, 16), jnp.float32),
        input_output_aliases={4: 0},
    )(t0, t1, u, vh, prev)


def kernel(x, edge_attr, edge_index, mp_fc0_w, mp_fc0_b, mp_out_w, mp_out_b,
           fc0_w, fc0_b, fc_out_w, fc_out_b):
    del edge_attr

    src = edge_index[0].astype(jnp.int32)
    dst = edge_index[1].astype(jnp.int32)
    pad_e = E_PAD - N_EDGES
    src2d = jnp.concatenate(
        [src, jnp.zeros((pad_e,), jnp.int32)]).reshape(N_CHUNKS, CH)
    dst2d = jnp.concatenate(
        [dst, jnp.full((pad_e,), N_NODES, jnp.int32)]).reshape(N_CHUNKS, CH)

    x_pad = jnp.pad(x, ((0, N_PAD - N_NODES), (0, 0)))

    W_xs = fc0_w[:, 0:128].T
    W_xd = fc0_w[:, 128:256].T
    W_as = fc0_w[:, 256:768].T
    W_ad = fc0_w[:, 768:1280].T
    V_h = fc_out_w[:, 0:128].T
    V_xs = fc_out_w[:, 128:256].T
    V_xd = fc_out_w[:, 256:384].T
    V_as = fc_out_w[:, 384:896].T
    V_ad = fc_out_w[:, 896:1408].T

    wcat_a = jnp.concatenate([W_as, V_as], axis=1)
    wcat_b = jnp.concatenate([W_ad, V_ad], axis=1)
    wx_a = jnp.concatenate([W_xs, V_xs], axis=1)
    wx_b = jnp.concatenate([W_xd, V_xd], axis=1)
    bias_b = jnp.concatenate([fc0_b, fc_out_b])[None, :]

    h2a, h2b, xwa, xwb = _dense_pre(
        x_pad, mp_fc0_w.T, mp_fc0_b[None, :], mp_out_w.T, mp_out_b[None, :],
        wcat_a, wcat_b, wx_a, wx_b, bias_b)

    pa = _sc_segsum(h2a, xwa, src2d, dst2d)
    pb = _sc_segsum(h2b, xwb, src2d, dst2d)
    srcE, dstE = src2d, dst2d
    th0a, th1a, ua = _sc_edge(pa, pb, srcE, dstE, 0)
    th0b, th1b, ub = _sc_edge(pa, pb, srcE, dstE, EH_CHUNKS)
    out1 = _final_half1(th0a, th1a, ua, V_h)
    return _final_half2(th0b, th1b, ub, V_h, out1)


# speedup vs baseline: 1.0570x; 1.0570x over previous
import jax
import jax.numpy as jnp
from jax import lax
from jax.experimental import pallas as pl
from jax.experimental.pallas import tpu as pltpu
from jax.experimental.pallas import tpu_sc as plsc

N_NODES = 10000
N_EDGES = 320000
D_FEAT = 128
HALF = 144

NS = 16

N_PAD = 10240
E_PAD = 327680

CH = 128
N_CHUNKS = E_PAD // CH
TC_CHUNKS = N_CHUNKS // NS
N_ACC = 10112
STRIPE = N_ACC // NS

MB = 1024
EB = 4000

_SC_PARAMS = pltpu.CompilerParams(use_tc_tiling_on_sc=False)


def _dense_pre_body(x_ref, w1_ref, b1_ref, w2_ref, b2_ref,
                    wcat_a_ref, wcat_b_ref, wx_a_ref, wx_b_ref, bias_b_ref,
                    h2a_ref, h2b_ref, xwa_ref, xwb_ref):
    x = x_ref[...]
    h1 = jnp.maximum(x @ w1_ref[...] + b1_ref[...], 0.0)
    h = h1 @ w2_ref[...] + b2_ref[...]
    h2a_ref[...] = h @ wcat_a_ref[...]
    h2b_ref[...] = h @ wcat_b_ref[...]
    xwa_ref[...] = x @ wx_a_ref[...]
    xwb_ref[...] = x @ wx_b_ref[...] + bias_b_ref[...]


def _dense_pre(x, w1, b1, w2, b2, wcat_a, wcat_b, wx_a, wx_b, bias_b):
    grid = (N_PAD // MB,)
    full = lambda shape: pl.BlockSpec(shape, lambda i: (0, 0))
    return pl.pallas_call(
        _dense_pre_body,
        grid=grid,
        in_specs=[
            pl.BlockSpec((MB, D_FEAT), lambda i: (i, 0)),
            full((D_FEAT, 128)), full((1, 128)),
            full((128, 512)), full((1, 512)),
            full((512, HALF)), full((512, HALF)),
            full((D_FEAT, HALF)), full((D_FEAT, HALF)), full((1, HALF)),
        ],
        out_specs=[pl.BlockSpec((MB, HALF), lambda i: (i, 0))] * 4,
        out_shape=[jax.ShapeDtypeStruct((N_PAD, HALF), jnp.float32)] * 4,
    )(x, w1, b1, w2, b2, wcat_a, wcat_b, wx_a, wx_b, bias_b)


IB = 8


def _sc_segsum_body(h2a_hbm, h2b_hbm, xwa_hbm, xwb_hbm, src_hbm, dst_hbm,
                    pa_hbm, pb_hbm,
                    idx_s, idx_d, rows0, rows1, sem0, sem1, acc):
    c = lax.axis_index("c")
    s = lax.axis_index("s")

    @pl.when(c == 0)
    def _():
        pltpu.sync_copy(xwa_hbm.at[pl.ds(s * STRIPE, STRIPE)],
                        acc.at[pl.ds(s * STRIPE, STRIPE)])

    @pl.when(c == 1)
    def _():
        pltpu.sync_copy(xwb_hbm.at[pl.ds(s * STRIPE, STRIPE)],
                        acc.at[pl.ds(s * STRIPE, STRIPE)])

    plsc.subcore_barrier()

    def gather(j, buf, sem):
        @pl.when(c == 0)
        def _():
            pltpu.async_copy(h2a_hbm.at[idx_s.at[j]], buf, sem)

        @pl.when(c == 1)
        def _():
            pltpu.async_copy(h2b_hbm.at[idx_s.at[j]], buf, sem)

    def wait_g(j, buf, sem):
        pltpu.make_async_copy(h2a_hbm.at[idx_s.at[j]], buf, sem).wait()

    def block(g, _):
        row0 = s * TC_CHUNKS + g * IB
        pltpu.sync_copy(src_hbm.at[pl.ds(row0, IB)], idx_s)
        pltpu.sync_copy(dst_hbm.at[pl.ds(row0, IB)], idx_d)
        gather(0, rows0, sem0)

        def pair(k, _):
            j0 = 2 * k
            gather(j0 + 1, rows1, sem1)
            wait_g(j0, rows0, sem0)
            pltpu.sync_copy(rows0, acc.at[idx_d.at[j0]], add=True)

            @pl.when(j0 + 2 < IB)
            def _():
                gather(j0 + 2, rows0, sem0)

            wait_g(j0, rows1, sem1)
            pltpu.sync_copy(rows1, acc.at[idx_d.at[j0 + 1]], add=True)
            return 0

        lax.fori_loop(0, IB // 2, pair, 0)
        return 0

    lax.fori_loop(0, TC_CHUNKS // IB, block, 0)
    plsc.subcore_barrier()

    @pl.when(c == 0)
    def _():
        pltpu.sync_copy(acc.at[pl.ds(s * STRIPE, STRIPE)],
                        pa_hbm.at[pl.ds(s * STRIPE, STRIPE)])

    @pl.when(c == 1)
    def _():
        pltpu.sync_copy(acc.at[pl.ds(s * STRIPE, STRIPE)],
                        pb_hbm.at[pl.ds(s * STRIPE, STRIPE)])


def _sc_segsum(h2a, h2b, xwa, xwb, src2d, dst2d):
    mesh = plsc.VectorSubcoreMesh(core_axis_name="c", subcore_axis_name="s",
                                  num_cores=2, num_subcores=NS)
    return pl.kernel(
        _sc_segsum_body,
        compiler_params=_SC_PARAMS,
        out_type=[jax.ShapeDtypeStruct((N_ACC, HALF), jnp.float32)] * 2,
        mesh=mesh,
        scratch_types=[
            pltpu.VMEM((IB, CH), jnp.int32),
            pltpu.VMEM((IB, CH), jnp.int32),
            pltpu.VMEM((CH, HALF), jnp.float32),
            pltpu.VMEM((CH, HALF), jnp.float32),
            pltpu.SemaphoreType.DMA,
            pltpu.SemaphoreType.DMA,
            pltpu.VMEM_SHARED((N_ACC, HALF), jnp.float32),
        ],
    )(h2a, h2b, xwa, xwb, src2d, dst2d)


CHE = 128
IBE = 8
EH = E_PAD // 2
EH_CHUNKS = EH // CHE
TE_CHUNKS = EH_CHUNKS // (2 * NS)


def _sc_edge_body(base_chunk,
                  pa_hbm, pb_hbm, src_hbm, dst_hbm,
                  th0_hbm, th1_hbm, u_hbm,
                  idx_s, idx_d, ra0, ra1, rb0, rb1, uv,
                  sa0, sa1, sb0, sb1):
    c = lax.axis_index("c")
    s = lax.axis_index("s")
    wid = c * NS + s

    def block(g, _):
        row0 = wid * TE_CHUNKS + g * IBE
        pltpu.sync_copy(src_hbm.at[pl.ds(base_chunk + row0, IBE)], idx_s)
        pltpu.sync_copy(dst_hbm.at[pl.ds(base_chunk + row0, IBE)], idx_d)

        def issue(j, ra, sa, rb, sb):
            pltpu.async_copy(pa_hbm.at[idx_s.at[j]], ra, sa)
            pltpu.async_copy(pb_hbm.at[idx_d.at[j]], rb, sb)

        def consume(j, ra, sa, rb, sb):
            base = (row0 + j) * CHE
            pltpu.make_async_copy(pa_hbm.at[idx_s.at[j]], ra, sa).wait()
            pltpu.make_async_copy(pb_hbm.at[idx_d.at[j]], rb, sb).wait()
            pltpu.sync_copy(ra.at[:, pl.ds(0, 128)], th0_hbm.at[pl.ds(base, CHE)])
            pltpu.sync_copy(rb.at[:, pl.ds(0, 128)], th1_hbm.at[pl.ds(base, CHE)])

            def uadd(e, _):
                uv[e, :] = ra[e, pl.ds(128, 16)] + rb[e, pl.ds(128, 16)]
                return 0

            lax.fori_loop(0, CHE, uadd, 0)
            pltpu.sync_copy(uv, u_hbm.at[pl.ds(base, CHE)])

        issue(0, ra0, sa0, rb0, sb0)

        def pair(k, _):
            j0 = 2 * k
            issue(j0 + 1, ra1, sa1, rb1, sb1)
            consume(j0, ra0, sa0, rb0, sb0)

            @pl.when(j0 + 2 < IBE)
            def _():
                issue(j0 + 2, ra0, sa0, rb0, sb0)

            consume(j0 + 1, ra1, sa1, rb1, sb1)
            return 0

        lax.fori_loop(0, IBE // 2, pair, 0)
        return 0

    lax.fori_loop(0, TE_CHUNKS // IBE, block, 0)


def _sc_edge(pa, pb, src2d, dst2d, base_chunk):
    import functools
    mesh = plsc.VectorSubcoreMesh(core_axis_name="c", subcore_axis_name="s",
                                  num_cores=2, num_subcores=NS)
    return pl.kernel(
        functools.partial(_sc_edge_body, base_chunk),
        compiler_params=_SC_PARAMS,
        out_type=[jax.ShapeDtypeStruct((EH, 128), jnp.float32),
                  jax.ShapeDtypeStruct((EH, 128), jnp.float32),
                  jax.ShapeDtypeStruct((EH, 16), jnp.float32)],
        mesh=mesh,
        scratch_types=[
            pltpu.VMEM((IBE, CHE), jnp.int32),
            pltpu.VMEM((IBE, CHE), jnp.int32),
            pltpu.VMEM((CHE, HALF), jnp.float32),
            pltpu.VMEM((CHE, HALF), jnp.float32),
            pltpu.VMEM((CHE, HALF), jnp.float32),
            pltpu.VMEM((CHE, HALF), jnp.float32),
            pltpu.VMEM((CHE, 16), jnp.float32),
            pltpu.SemaphoreType.DMA,
            pltpu.SemaphoreType.DMA,
            pltpu.SemaphoreType.DMA,
            pltpu.SemaphoreType.DMA,
        ],
    )(pa, pb, src2d, dst2d)


FB = 2560


def _final_body(t0_ref, t1_ref, u_ref, vh_ref, o_ref):
    hid = jnp.maximum(t0_ref[...] + t1_ref[...], 0.0)
    o_ref[...] = hid @ vh_ref[...] + u_ref[...]


def _final_half1(t0, t1, u, vh):
    return pl.pallas_call(
        _final_body,
        grid=(EH // FB,),
        in_specs=[
            pl.BlockSpec((FB, 128), lambda i: (i, 0)),
            pl.BlockSpec((FB, 128), lambda i: (i, 0)),
            pl.BlockSpec((FB, 16), lambda i: (i, 0)),
            pl.BlockSpec((128, 16), lambda i: (0, 0)),
        ],
        out_specs=pl.BlockSpec((FB, 16), lambda i: (i, 0)),
        out_shape=jax.ShapeDtypeStruct((N_EDGES, 16), jnp.float32),
    )(t0, t1, u, vh)


def _final_half2_body(t0_ref, t1_ref, u_ref, vh_ref, prev_ref, o_ref):
    del prev_ref
    hid = jnp.maximum(t0_ref[...] + t1_ref[...], 0.0)
    o_ref[...] = hid @ vh_ref[...] + u_ref[...]


def _final_half2(t0, t1, u, vh, prev):
    n_blocks = (N_EDGES - EH) // FB
    base = EH // FB
    return pl.pallas_call(
        _final_half2_body,
        grid=(n_blocks,),
        in_specs=[
            pl.BlockSpec((FB, 128), lambda i: (i, 0)),
            pl.BlockSpec((FB, 128), lambda i: (i, 0)),
            pl.BlockSpec((FB, 16), lambda i: (i, 0)),
            pl.BlockSpec((128, 16), lambda i: (0, 0)),
            pl.BlockSpec(memory_space=pl.ANY),
        ],
        out_specs=pl.BlockSpec((FB, 16), lambda i: (base + i, 0)),
        out_shape=jax.ShapeDtypeStruct((N_EDGES, 16), jnp.float32),
        input_output_aliases={4: 0},
    )(t0, t1, u, vh, prev)


def kernel(x, edge_attr, edge_index, mp_fc0_w, mp_fc0_b, mp_out_w, mp_out_b,
           fc0_w, fc0_b, fc_out_w, fc_out_b):
    del edge_attr

    src = edge_index[0].astype(jnp.int32)
    dst = edge_index[1].astype(jnp.int32)
    pad_e = E_PAD - N_EDGES
    src2d = jnp.concatenate(
        [src, jnp.zeros((pad_e,), jnp.int32)]).reshape(N_CHUNKS, CH)
    dst2d = jnp.concatenate(
        [dst, jnp.full((pad_e,), N_NODES, jnp.int32)]).reshape(N_CHUNKS, CH)

    x_pad = jnp.pad(x, ((0, N_PAD - N_NODES), (0, 0)))

    W_xs = fc0_w[:, 0:128].T
    W_xd = fc0_w[:, 128:256].T
    W_as = fc0_w[:, 256:768].T
    W_ad = fc0_w[:, 768:1280].T
    V_h = fc_out_w[:, 0:128].T
    V_xs = fc_out_w[:, 128:256].T
    V_xd = fc_out_w[:, 256:384].T
    V_as = fc_out_w[:, 384:896].T
    V_ad = fc_out_w[:, 896:1408].T

    wcat_a = jnp.concatenate([W_as, V_as], axis=1)
    wcat_b = jnp.concatenate([W_ad, V_ad], axis=1)
    wx_a = jnp.concatenate([W_xs, V_xs], axis=1)
    wx_b = jnp.concatenate([W_xd, V_xd], axis=1)
    bias_b = jnp.concatenate([fc0_b, fc_out_b])[None, :]

    h2a, h2b, xwa, xwb = _dense_pre(
        x_pad, mp_fc0_w.T, mp_fc0_b[None, :], mp_out_w.T, mp_out_b[None, :],
        wcat_a, wcat_b, wx_a, wx_b, bias_b)

    pa, pb = _sc_segsum(h2a, h2b, xwa, xwb, src2d, dst2d)
    srcE, dstE = src2d, dst2d
    th0a, th1a, ua = _sc_edge(pa, pb, srcE, dstE, 0)
    th0b, th1b, ub = _sc_edge(pa, pb, srcE, dstE, EH_CHUNKS)
    out1 = _final_half1(th0a, th1a, ua, V_h)
    return _final_half2(th0b, th1b, ub, V_h, out1)


# speedup vs baseline: 2.5022x; 2.3672x over previous
import jax
import jax.numpy as jnp
from jax import lax
from jax.experimental import pallas as pl
from jax.experimental.pallas import tpu as pltpu
from jax.experimental.pallas import tpu_sc as plsc

N_NODES = 10000
N_EDGES = 320000
D_FEAT = 128
HALF = 144

NS = 16

N_PAD = 10240
E_PAD = 327680

CH = 80
N_CHUNKS = E_PAD // CH
TC_CHUNKS = N_CHUNKS // NS
N_ACC = N_PAD
STRIPE = N_ACC // NS

MB = 1024
EB = 4000

_SC_PARAMS = pltpu.CompilerParams(use_tc_tiling_on_sc=False)


def _dense_pre_body(x_ref, w1_ref, b1_ref, w2_ref, b2_ref,
                    wcat_a_ref, wcat_b_ref, wx_a_ref, wx_b_ref, bias_b_ref,
                    h2a_ref, h2b_ref, xwa_ref, xwb_ref):
    x = x_ref[...]
    h1 = jnp.maximum(x @ w1_ref[...] + b1_ref[...], 0.0)
    h = h1 @ w2_ref[...] + b2_ref[...]
    h2a_ref[...] = h @ wcat_a_ref[...]
    h2b_ref[...] = h @ wcat_b_ref[...]
    xwa_ref[...] = x @ wx_a_ref[...]
    xwb_ref[...] = x @ wx_b_ref[...] + bias_b_ref[...]


def _dense_pre(x, w1, b1, w2, b2, wcat_a, wcat_b, wx_a, wx_b, bias_b):
    grid = (N_PAD // MB,)
    full = lambda shape: pl.BlockSpec(shape, lambda i: (0, 0))
    return pl.pallas_call(
        _dense_pre_body,
        grid=grid,
        in_specs=[
            pl.BlockSpec((MB, D_FEAT), lambda i: (i, 0)),
            full((D_FEAT, 128)), full((1, 128)),
            full((128, 512)), full((1, 512)),
            full((512, HALF)), full((512, HALF)),
            full((D_FEAT, HALF)), full((D_FEAT, HALF)), full((1, HALF)),
        ],
        out_specs=[pl.BlockSpec((MB, HALF), lambda i: (i, 0))] * 4,
        out_shape=[jax.ShapeDtypeStruct((N_PAD, HALF), jnp.float32)] * 4,
    )(x, w1, b1, w2, b2, wcat_a, wcat_b, wx_a, wx_b, bias_b)


IB = 64


def _sc_segsum_body(h2a_hbm, h2b_hbm, xwa_hbm, xwb_hbm, src_hbm, dst_hbm,
                    pa_hbm, pb_hbm,
                    idx_s, idx_d, rows0, rows1, sem0, sem1, acc):
    c = lax.axis_index("c")
    s = lax.axis_index("s")

    @pl.when(c == 0)
    def _():
        pltpu.sync_copy(xwa_hbm.at[pl.ds(s * STRIPE, STRIPE)],
                        acc.at[pl.ds(s * STRIPE, STRIPE)])

    @pl.when(c == 1)
    def _():
        pltpu.sync_copy(xwb_hbm.at[pl.ds(s * STRIPE, STRIPE)],
                        acc.at[pl.ds(s * STRIPE, STRIPE)])

    plsc.subcore_barrier()

    def gather(j, buf, sem):
        @pl.when(c == 0)
        def _():
            pltpu.async_copy(h2a_hbm.at[idx_s.at[j]], buf, sem)

        @pl.when(c == 1)
        def _():
            pltpu.async_copy(h2b_hbm.at[idx_s.at[j]], buf, sem)

    def wait_g(j, buf, sem):
        pltpu.make_async_copy(h2a_hbm.at[idx_s.at[j]], buf, sem).wait()

    def block(g, _):
        row0 = s * TC_CHUNKS + g * IB
        pltpu.sync_copy(src_hbm.at[pl.ds(row0, IB)], idx_s)
        pltpu.sync_copy(dst_hbm.at[pl.ds(row0, IB)], idx_d)
        gather(0, rows0, sem0)

        def pair(k, _):
            j0 = 2 * k
            gather(j0 + 1, rows1, sem1)
            wait_g(j0, rows0, sem0)
            pltpu.sync_copy(rows0, acc.at[idx_d.at[j0]], add=True)

            @pl.when(j0 + 2 < IB)
            def _():
                gather(j0 + 2, rows0, sem0)

            wait_g(j0, rows1, sem1)
            pltpu.sync_copy(rows1, acc.at[idx_d.at[j0 + 1]], add=True)
            return 0

        lax.fori_loop(0, IB // 2, pair, 0)
        return 0

    lax.fori_loop(0, TC_CHUNKS // IB, block, 0)
    plsc.subcore_barrier()

    @pl.when(c == 0)
    def _():
        pltpu.sync_copy(acc.at[pl.ds(s * STRIPE, STRIPE)],
                        pa_hbm.at[pl.ds(s * STRIPE, STRIPE)])

    @pl.when(c == 1)
    def _():
        pltpu.sync_copy(acc.at[pl.ds(s * STRIPE, STRIPE)],
                        pb_hbm.at[pl.ds(s * STRIPE, STRIPE)])


def _sc_segsum(h2a, h2b, xwa, xwb, src2d, dst2d):
    mesh = plsc.VectorSubcoreMesh(core_axis_name="c", subcore_axis_name="s",
                                  num_cores=2, num_subcores=NS)
    return pl.kernel(
        _sc_segsum_body,
        compiler_params=_SC_PARAMS,
        out_type=[jax.ShapeDtypeStruct((N_ACC, HALF), jnp.float32)] * 2,
        mesh=mesh,
        scratch_types=[
            pltpu.VMEM((IB, CH), jnp.int32),
            pltpu.VMEM((IB, CH), jnp.int32),
            pltpu.VMEM((CH, HALF), jnp.float32),
            pltpu.VMEM((CH, HALF), jnp.float32),
            pltpu.SemaphoreType.DMA,
            pltpu.SemaphoreType.DMA,
            pltpu.VMEM_SHARED((N_ACC, HALF), jnp.float32),
        ],
    )(h2a, h2b, xwa, xwb, src2d, dst2d)


CHE = 128
IBE = 8
EH = E_PAD // 2
EH_CHUNKS = EH // CHE
TE_CHUNKS = EH_CHUNKS // (2 * NS)


def _sc_edge_body(base_chunk,
                  pa_hbm, pb_hbm, src_hbm, dst_hbm,
                  th0_hbm, th1_hbm, u_hbm,
                  idx_s, idx_d, ra0, ra1, rb0, rb1, uv,
                  sa0, sa1, sb0, sb1):
    c = lax.axis_index("c")
    s = lax.axis_index("s")
    wid = c * NS + s

    def block(g, _):
        row0 = wid * TE_CHUNKS + g * IBE
        pltpu.sync_copy(src_hbm.at[pl.ds(base_chunk + row0, IBE)], idx_s)
        pltpu.sync_copy(dst_hbm.at[pl.ds(base_chunk + row0, IBE)], idx_d)

        def issue(j, ra, sa, rb, sb):
            pltpu.async_copy(pa_hbm.at[idx_s.at[j]], ra, sa)
            pltpu.async_copy(pb_hbm.at[idx_d.at[j]], rb, sb)

        def consume(j, ra, sa, rb, sb):
            base = (row0 + j) * CHE
            pltpu.make_async_copy(pa_hbm.at[idx_s.at[j]], ra, sa).wait()
            pltpu.make_async_copy(pb_hbm.at[idx_d.at[j]], rb, sb).wait()
            pltpu.sync_copy(ra.at[:, pl.ds(0, 128)], th0_hbm.at[pl.ds(base, CHE)])
            pltpu.sync_copy(rb.at[:, pl.ds(0, 128)], th1_hbm.at[pl.ds(base, CHE)])

            def uadd(e, _):
                uv[e, :] = ra[e, pl.ds(128, 16)] + rb[e, pl.ds(128, 16)]
                return 0

            lax.fori_loop(0, CHE, uadd, 0)
            pltpu.sync_copy(uv, u_hbm.at[pl.ds(base, CHE)])

        issue(0, ra0, sa0, rb0, sb0)

        def pair(k, _):
            j0 = 2 * k
            issue(j0 + 1, ra1, sa1, rb1, sb1)
            consume(j0, ra0, sa0, rb0, sb0)

            @pl.when(j0 + 2 < IBE)
            def _():
                issue(j0 + 2, ra0, sa0, rb0, sb0)

            consume(j0 + 1, ra1, sa1, rb1, sb1)
            return 0

        lax.fori_loop(0, IBE // 2, pair, 0)
        return 0

    lax.fori_loop(0, TE_CHUNKS // IBE, block, 0)


def _sc_edge(pa, pb, src2d, dst2d, base_chunk):
    import functools
    mesh = plsc.VectorSubcoreMesh(core_axis_name="c", subcore_axis_name="s",
                                  num_cores=2, num_subcores=NS)
    return pl.kernel(
        functools.partial(_sc_edge_body, base_chunk),
        compiler_params=_SC_PARAMS,
        out_type=[jax.ShapeDtypeStruct((EH, 128), jnp.float32),
                  jax.ShapeDtypeStruct((EH, 128), jnp.float32),
                  jax.ShapeDtypeStruct((EH, 16), jnp.float32)],
        mesh=mesh,
        scratch_types=[
            pltpu.VMEM((IBE, CHE), jnp.int32),
            pltpu.VMEM((IBE, CHE), jnp.int32),
            pltpu.VMEM((CHE, HALF), jnp.float32),
            pltpu.VMEM((CHE, HALF), jnp.float32),
            pltpu.VMEM((CHE, HALF), jnp.float32),
            pltpu.VMEM((CHE, HALF), jnp.float32),
            pltpu.VMEM((CHE, 16), jnp.float32),
            pltpu.SemaphoreType.DMA,
            pltpu.SemaphoreType.DMA,
            pltpu.SemaphoreType.DMA,
            pltpu.SemaphoreType.DMA,
        ],
    )(pa, pb, src2d, dst2d)


FB = 2560


def _final_body(t0_ref, t1_ref, u_ref, vh_ref, o_ref):
    hid = jnp.maximum(t0_ref[...] + t1_ref[...], 0.0)
    o_ref[...] = hid @ vh_ref[...] + u_ref[...]


def _final_half1(t0, t1, u, vh):
    return pl.pallas_call(
        _final_body,
        grid=(EH // FB,),
        in_specs=[
            pl.BlockSpec((FB, 128), lambda i: (i, 0)),
            pl.BlockSpec((FB, 128), lambda i: (i, 0)),
            pl.BlockSpec((FB, 16), lambda i: (i, 0)),
            pl.BlockSpec((128, 16), lambda i: (0, 0)),
        ],
        out_specs=pl.BlockSpec((FB, 16), lambda i: (i, 0)),
        out_shape=jax.ShapeDtypeStruct((N_EDGES, 16), jnp.float32),
    )(t0, t1, u, vh)


def _final_half2_body(t0_ref, t1_ref, u_ref, vh_ref, prev_ref, o_ref):
    del prev_ref
    hid = jnp.maximum(t0_ref[...] + t1_ref[...], 0.0)
    o_ref[...] = hid @ vh_ref[...] + u_ref[...]


def _final_half2(t0, t1, u, vh, prev):
    n_blocks = (N_EDGES - EH) // FB
    base = EH // FB
    return pl.pallas_call(
        _final_half2_body,
        grid=(n_blocks,),
        in_specs=[
            pl.BlockSpec((FB, 128), lambda i: (i, 0)),
            pl.BlockSpec((FB, 128), lambda i: (i, 0)),
            pl.BlockSpec((FB, 16), lambda i: (i, 0)),
            pl.BlockSpec((128, 16), lambda i: (0, 0)),
            pl.BlockSpec(memory_space=pl.ANY),
        ],
        out_specs=pl.BlockSpec((FB, 16), lambda i: (base + i, 0)),
        out_shape=jax.ShapeDtypeStruct((N_EDGES, 16), jnp.float32),
        input_output_aliases={4: 0},
    )(t0, t1, u, vh, prev)


def kernel(x, edge_attr, edge_index, mp_fc0_w, mp_fc0_b, mp_out_w, mp_out_b,
           fc0_w, fc0_b, fc_out_w, fc_out_b):
    del edge_attr

    src = edge_index[0].astype(jnp.int32)
    dst = edge_index[1].astype(jnp.int32)
    pad_e = E_PAD - N_EDGES
    pad_ar = jnp.arange(pad_e, dtype=jnp.int32)
    src_pad = jnp.concatenate([src, pad_ar % N_NODES])
    dst_pad = jnp.concatenate([dst, N_NODES + pad_ar % (N_PAD - N_NODES)])
    src2d = src_pad.reshape(N_CHUNKS, CH)
    dst2d = dst_pad.reshape(N_CHUNKS, CH)

    x_pad = jnp.pad(x, ((0, N_PAD - N_NODES), (0, 0)))

    W_xs = fc0_w[:, 0:128].T
    W_xd = fc0_w[:, 128:256].T
    W_as = fc0_w[:, 256:768].T
    W_ad = fc0_w[:, 768:1280].T
    V_h = fc_out_w[:, 0:128].T
    V_xs = fc_out_w[:, 128:256].T
    V_xd = fc_out_w[:, 256:384].T
    V_as = fc_out_w[:, 384:896].T
    V_ad = fc_out_w[:, 896:1408].T

    wcat_a = jnp.concatenate([W_as, V_as], axis=1)
    wcat_b = jnp.concatenate([W_ad, V_ad], axis=1)
    wx_a = jnp.concatenate([W_xs, V_xs], axis=1)
    wx_b = jnp.concatenate([W_xd, V_xd], axis=1)
    bias_b = jnp.concatenate([fc0_b, fc_out_b])[None, :]

    h2a, h2b, xwa, xwb = _dense_pre(
        x_pad, mp_fc0_w.T, mp_fc0_b[None, :], mp_out_w.T, mp_out_b[None, :],
        wcat_a, wcat_b, wx_a, wx_b, bias_b)

    pa, pb = _sc_segsum(h2a, h2b, xwa, xwb, src2d, dst2d)
    srcE = src_pad.reshape(E_PAD // CHE, CHE)
    dstE = dst_pad.reshape(E_PAD // CHE, CHE)
    th0a, th1a, ua = _sc_edge(pa, pb, srcE, dstE, 0)
    th0b, th1b, ub = _sc_edge(pa, pb, srcE, dstE, EH_CHUNKS)
    out1 = _final_half1(th0a, th1a, ua, V_h)
    return _final_half2(th0b, th1b, ub, V_h, out1)


# speedup vs baseline: 2.5043x; 1.0008x over previous
import functools

import jax
import jax.numpy as jnp
from jax import lax
from jax.experimental import pallas as pl
from jax.experimental.pallas import tpu as pltpu
from jax.experimental.pallas import tpu_sc as plsc

N_NODES = 10000
N_EDGES = 320000
D_FEAT = 128
HALF = 144

NS = 16

N_PAD = 10240
E_PAD = 327680

CH = 80
N_CHUNKS = E_PAD // CH
TC_CHUNKS = N_CHUNKS // NS
N_ACC = N_PAD
STRIPE = N_ACC // NS

MB = 1024
_SC_PARAMS = pltpu.CompilerParams(use_tc_tiling_on_sc=False)


def _dense_pre_body(x_ref, w1_ref, b1_ref, w2_ref, b2_ref,
                    wcat_a_ref, wcat_b_ref, wx_a_ref, wx_b_ref, bias_b_ref,
                    h2a_ref, h2b_ref, xwa_ref, xwb_ref):
    x = x_ref[...]
    h1 = jnp.maximum(x @ w1_ref[...] + b1_ref[...], 0.0)
    h = h1 @ w2_ref[...] + b2_ref[...]
    h2a_ref[...] = h @ wcat_a_ref[...]
    h2b_ref[...] = h @ wcat_b_ref[...]
    xwa_ref[...] = x @ wx_a_ref[...]
    xwb_ref[...] = x @ wx_b_ref[...] + bias_b_ref[...]


def _dense_pre(x, w1, b1, w2, b2, wcat_a, wcat_b, wx_a, wx_b, bias_b):
    grid = (N_PAD // MB,)
    full = lambda shape: pl.BlockSpec(shape, lambda i: (0, 0))
    return pl.pallas_call(
        _dense_pre_body,
        grid=grid,
        in_specs=[
            pl.BlockSpec((MB, D_FEAT), lambda i: (i, 0)),
            full((D_FEAT, 128)), full((1, 128)),
            full((128, 512)), full((1, 512)),
            full((512, HALF)), full((512, HALF)),
            full((D_FEAT, HALF)), full((D_FEAT, HALF)), full((1, HALF)),
        ],
        out_specs=[pl.BlockSpec((MB, HALF), lambda i: (i, 0))] * 4,
        out_shape=[jax.ShapeDtypeStruct((N_PAD, HALF), jnp.float32)] * 4,
    )(x, w1, b1, w2, b2, wcat_a, wcat_b, wx_a, wx_b, bias_b)


IB = 64


def _sc_segsum_body(h2a_hbm, h2b_hbm, xwa_hbm, xwb_hbm, src_hbm, dst_hbm,
                    pa_hbm, pb_hbm,
                    idx_s, idx_d, rows0, rows1, sem0, sem1, acc):
    c = lax.axis_index("c")
    s = lax.axis_index("s")

    @pl.when(c == 0)
    def _():
        pltpu.sync_copy(xwa_hbm.at[pl.ds(s * STRIPE, STRIPE)],
                        acc.at[pl.ds(s * STRIPE, STRIPE)])

    @pl.when(c == 1)
    def _():
        pltpu.sync_copy(xwb_hbm.at[pl.ds(s * STRIPE, STRIPE)],
                        acc.at[pl.ds(s * STRIPE, STRIPE)])

    plsc.subcore_barrier()

    def gather(j, buf, sem):
        @pl.when(c == 0)
        def _():
            pltpu.async_copy(h2a_hbm.at[idx_s.at[j]], buf, sem)

        @pl.when(c == 1)
        def _():
            pltpu.async_copy(h2b_hbm.at[idx_s.at[j]], buf, sem)

    def wait_g(j, buf, sem):
        pltpu.make_async_copy(h2a_hbm.at[idx_s.at[j]], buf, sem).wait()

    def block(g, _):
        row0 = s * TC_CHUNKS + g * IB
        pltpu.sync_copy(src_hbm.at[pl.ds(row0, IB)], idx_s)
        pltpu.sync_copy(dst_hbm.at[pl.ds(row0, IB)], idx_d)
        gather(0, rows0, sem0)

        def pair(k, _):
            j0 = 2 * k
            gather(j0 + 1, rows1, sem1)
            wait_g(j0, rows0, sem0)
            pltpu.sync_copy(rows0, acc.at[idx_d.at[j0]], add=True)

            @pl.when(j0 + 2 < IB)
            def _():
                gather(j0 + 2, rows0, sem0)

            wait_g(j0, rows1, sem1)
            pltpu.sync_copy(rows1, acc.at[idx_d.at[j0 + 1]], add=True)
            return 0

        lax.fori_loop(0, IB // 2, pair, 0)
        return 0

    lax.fori_loop(0, TC_CHUNKS // IB, block, 0)
    plsc.subcore_barrier()

    @pl.when(c == 0)
    def _():
        pltpu.sync_copy(acc.at[pl.ds(s * STRIPE, STRIPE)],
                        pa_hbm.at[pl.ds(s * STRIPE, STRIPE)])

    @pl.when(c == 1)
    def _():
        pltpu.sync_copy(acc.at[pl.ds(s * STRIPE, STRIPE)],
                        pb_hbm.at[pl.ds(s * STRIPE, STRIPE)])


def _sc_segsum(h2a, h2b, xwa, xwb, src2d, dst2d):
    mesh = plsc.VectorSubcoreMesh(core_axis_name="c", subcore_axis_name="s",
                                  num_cores=2, num_subcores=NS)
    return pl.kernel(
        _sc_segsum_body,
        compiler_params=_SC_PARAMS,
        out_type=[jax.ShapeDtypeStruct((N_ACC, HALF), jnp.float32)] * 2,
        mesh=mesh,
        scratch_types=[
            pltpu.VMEM((IB, CH), jnp.int32),
            pltpu.VMEM((IB, CH), jnp.int32),
            pltpu.VMEM((CH, HALF), jnp.float32),
            pltpu.VMEM((CH, HALF), jnp.float32),
            pltpu.SemaphoreType.DMA,
            pltpu.SemaphoreType.DMA,
            pltpu.VMEM_SHARED((N_ACC, HALF), jnp.float32),
        ],
    )(h2a, h2b, xwa, xwb, src2d, dst2d)


CHE = 128
IBE = 8
EH = E_PAD // 2
EH_CHUNKS = EH // CHE
TE_CHUNKS = EH_CHUNKS // (2 * NS)


def _sc_edge_body(base_chunk,
                  pa_hbm, pb_hbm, src_hbm, dst_hbm,
                  th0_hbm, th1_hbm, u_hbm,
                  idx_s, idx_d, ra0, ra1, rb0, rb1, uv,
                  sa0, sa1, sb0, sb1):
    c = lax.axis_index("c")
    s = lax.axis_index("s")
    wid = c * NS + s

    def block(g, _):
        row0 = wid * TE_CHUNKS + g * IBE
        pltpu.sync_copy(src_hbm.at[pl.ds(base_chunk + row0, IBE)], idx_s)
        pltpu.sync_copy(dst_hbm.at[pl.ds(base_chunk + row0, IBE)], idx_d)

        def issue(j, ra, sa, rb, sb):
            pltpu.async_copy(pa_hbm.at[idx_s.at[j]], ra, sa)
            pltpu.async_copy(pb_hbm.at[idx_d.at[j]], rb, sb)

        def consume(j, ra, sa, rb, sb):
            base = (row0 + j) * CHE
            pltpu.make_async_copy(pa_hbm.at[idx_s.at[j]], ra, sa).wait()
            pltpu.make_async_copy(pb_hbm.at[idx_d.at[j]], rb, sb).wait()
            pltpu.sync_copy(ra.at[:, pl.ds(0, 128)], th0_hbm.at[pl.ds(base, CHE)])
            pltpu.sync_copy(rb.at[:, pl.ds(0, 128)], th1_hbm.at[pl.ds(base, CHE)])

            def uadd(e, _):
                uv[e, :] = ra[e, pl.ds(128, 16)] + rb[e, pl.ds(128, 16)]
                return 0

            lax.fori_loop(0, CHE, uadd, 0)
            pltpu.sync_copy(uv, u_hbm.at[pl.ds(base, CHE)])

        issue(0, ra0, sa0, rb0, sb0)

        def pair(k, _):
            j0 = 2 * k
            issue(j0 + 1, ra1, sa1, rb1, sb1)
            consume(j0, ra0, sa0, rb0, sb0)

            @pl.when(j0 + 2 < IBE)
            def _():
                issue(j0 + 2, ra0, sa0, rb0, sb0)

            consume(j0 + 1, ra1, sa1, rb1, sb1)
            return 0

        lax.fori_loop(0, IBE // 2, pair, 0)
        return 0

    lax.fori_loop(0, TE_CHUNKS // IBE, block, 0)


def _sc_edge(pa, pb, src2d, dst2d, base_chunk):
    mesh = plsc.VectorSubcoreMesh(core_axis_name="c", subcore_axis_name="s",
                                  num_cores=2, num_subcores=NS)
    return pl.kernel(
        functools.partial(_sc_edge_body, base_chunk),
        compiler_params=_SC_PARAMS,
        out_type=[jax.ShapeDtypeStruct((EH, 128), jnp.float32),
                  jax.ShapeDtypeStruct((EH, 128), jnp.float32),
                  jax.ShapeDtypeStruct((EH, 16), jnp.float32)],
        mesh=mesh,
        scratch_types=[
            pltpu.VMEM((IBE, CHE), jnp.int32),
            pltpu.VMEM((IBE, CHE), jnp.int32),
            pltpu.VMEM((CHE, HALF), jnp.float32),
            pltpu.VMEM((CHE, HALF), jnp.float32),
            pltpu.VMEM((CHE, HALF), jnp.float32),
            pltpu.VMEM((CHE, HALF), jnp.float32),
            pltpu.VMEM((CHE, 16), jnp.float32),
            pltpu.SemaphoreType.DMA,
            pltpu.SemaphoreType.DMA,
            pltpu.SemaphoreType.DMA,
            pltpu.SemaphoreType.DMA,
        ],
    )(pa, pb, src2d, dst2d)


FB = 2560


def _final_body(t0_ref, t1_ref, u_ref, vh_ref, o_ref):
    hid = jnp.maximum(t0_ref[...] + t1_ref[...], 0.0)
    o_ref[...] = hid @ vh_ref[...] + u_ref[...]


def _final_half1(t0, t1, u, vh):
    return pl.pallas_call(
        _final_body,
        grid=(EH // FB,),
        in_specs=[
            pl.BlockSpec((FB, 128), lambda i: (i, 0)),
            pl.BlockSpec((FB, 128), lambda i: (i, 0)),
            pl.BlockSpec((FB, 16), lambda i: (i, 0)),
            pl.BlockSpec((128, 16), lambda i: (0, 0)),
        ],
        out_specs=pl.BlockSpec((FB, 16), lambda i: (i, 0)),
        out_shape=jax.ShapeDtypeStruct((N_EDGES, 16), jnp.float32),
    )(t0, t1, u, vh)


def _final_half2_body(t0_ref, t1_ref, u_ref, vh_ref, prev_ref, o_ref):
    del prev_ref
    hid = jnp.maximum(t0_ref[...] + t1_ref[...], 0.0)
    o_ref[...] = hid @ vh_ref[...] + u_ref[...]


def _final_half2(t0, t1, u, vh, prev):
    n_blocks = (N_EDGES - EH) // FB
    base = EH // FB
    return pl.pallas_call(
        _final_half2_body,
        grid=(n_blocks,),
        in_specs=[
            pl.BlockSpec((FB, 128), lambda i: (i, 0)),
            pl.BlockSpec((FB, 128), lambda i: (i, 0)),
            pl.BlockSpec((FB, 16), lambda i: (i, 0)),
            pl.BlockSpec((128, 16), lambda i: (0, 0)),
            pl.BlockSpec(memory_space=pl.ANY),
        ],
        out_specs=pl.BlockSpec((FB, 16), lambda i: (base + i, 0)),
        out_shape=jax.ShapeDtypeStruct((N_EDGES, 16), jnp.float32),
        input_output_aliases={4: 0},
    )(t0, t1, u, vh, prev)


def kernel(x, edge_attr, edge_index, mp_fc0_w, mp_fc0_b, mp_out_w, mp_out_b,
           fc0_w, fc0_b, fc_out_w, fc_out_b):
    del edge_attr

    src = edge_index[0].astype(jnp.int32)
    dst = edge_index[1].astype(jnp.int32)
    pad_e = E_PAD - N_EDGES
    pad_ar = jnp.arange(pad_e, dtype=jnp.int32)
    src_pad = jnp.concatenate([src, pad_ar % N_NODES])
    dst_pad = jnp.concatenate([dst, N_NODES + pad_ar % (N_PAD - N_NODES)])
    src2d = src_pad.reshape(N_CHUNKS, CH)
    dst2d = dst_pad.reshape(N_CHUNKS, CH)

    x_pad = jnp.pad(x, ((0, N_PAD - N_NODES), (0, 0)))

    W_xs = fc0_w[:, 0:128].T
    W_xd = fc0_w[:, 128:256].T
    W_as = fc0_w[:, 256:768].T
    W_ad = fc0_w[:, 768:1280].T
    V_h = fc_out_w[:, 0:128].T
    V_xs = fc_out_w[:, 128:256].T
    V_xd = fc_out_w[:, 256:384].T
    V_as = fc_out_w[:, 384:896].T
    V_ad = fc_out_w[:, 896:1408].T

    wcat_a = jnp.concatenate([W_as, V_as], axis=1)
    wcat_b = jnp.concatenate([W_ad, V_ad], axis=1)
    wx_a = jnp.concatenate([W_xs, V_xs], axis=1)
    wx_b = jnp.concatenate([W_xd, V_xd], axis=1)
    bias_b = jnp.concatenate([fc0_b, fc_out_b])[None, :]

    h2a, h2b, xwa, xwb = _dense_pre(
        x_pad, mp_fc0_w.T, mp_fc0_b[None, :], mp_out_w.T, mp_out_b[None, :],
        wcat_a, wcat_b, wx_a, wx_b, bias_b)

    pa, pb = _sc_segsum(h2a, h2b, xwa, xwb, src2d, dst2d)
    srcE = src_pad.reshape(E_PAD // CHE, CHE)
    dstE = dst_pad.reshape(E_PAD // CHE, CHE)
    th0a, th1a, ua = _sc_edge(pa, pb, srcE, dstE, 0)
    th0b, th1b, ub = _sc_edge(pa, pb, srcE, dstE, EH_CHUNKS)
    out1 = _final_half1(th0a, th1a, ua, V_h)
    return _final_half2(th0b, th1b, ub, V_h, out1)
